# baseline (speedup 1.0000x reference)
import functools
import math

import jax
import jax.numpy as jnp
from jax import lax
from jax.experimental import pallas as pl
from jax.experimental.pallas import tpu as pltpu
from jax.experimental.pallas import tpu_sc as plsc

_VOCAB = 1000000
_EMB = 64
_S = 200
_B = 4096
_N = _B * _S
_CHUNK = 128
_NC = 2
_NS = 16
_NW = _NC * _NS
_PER_W = _N // _NW
_CHUNKS_PER_W = _PER_W // _CHUNK
_G = 4
_SCALE = math.sqrt(_EMB)


def _make_kernel():
    mesh = plsc.VectorSubcoreMesh(core_axis_name="c", subcore_axis_name="s")

    @functools.partial(
        pl.kernel,
        mesh=mesh,
        out_type=jax.ShapeDtypeStruct((_N, _EMB), jnp.float32),
        compiler_params=pltpu.CompilerParams(use_tc_tiling_on_sc=False),
        scratch_types=[
            pltpu.VMEM((_CHUNKS_PER_W, _CHUNK), jnp.int32),
            pltpu.VMEM((2 * _S, _EMB), jnp.float32),
            pltpu.VMEM((_G, _CHUNK, _EMB), jnp.float32),
            pltpu.VMEM((2, _CHUNK, _EMB), jnp.float32),
            pltpu.SemaphoreType.DMA((_G,)),
            pltpu.SemaphoreType.DMA((2,)),
        ],
    )
    def k(x_hbm, tok_hbm, pos_hbm, out_hbm, idx_v, pos_v, inb, outb, gsem,
          ssem):
        wid = lax.axis_index("s") * _NC + lax.axis_index("c")
        cbase = wid * _CHUNKS_PER_W
        pltpu.sync_copy(x_hbm.at[pl.ds(cbase, _CHUNKS_PER_W)], idx_v)
        pltpu.sync_copy(pos_hbm.at[pl.ds(0, _S)], pos_v.at[pl.ds(0, _S)])
        pltpu.sync_copy(pos_hbm.at[pl.ds(0, _S)], pos_v.at[pl.ds(_S, _S)])

        def start_gather(j, b):
            pltpu.make_async_copy(
                tok_hbm.at[idx_v.at[j]], inb.at[b], gsem.at[b]).start()

        def wait_gather(b):
            pltpu.make_async_copy(
                tok_hbm.at[idx_v.at[0]], inb.at[b], gsem.at[b]).wait()

        def start_scatter(j, b):
            pltpu.make_async_copy(
                outb.at[b],
                out_hbm.at[pl.ds((cbase + j) * _CHUNK, _CHUNK)],
                ssem.at[b]).start()

        def wait_scatter(b):
            pltpu.make_async_copy(
                outb.at[b],
                out_hbm.at[pl.ds(cbase * _CHUNK, _CHUNK)],
                ssem.at[b]).wait()

        def compute(j, b, ob):
            base = lax.rem(j * _CHUNK, _S)

            def row_body(i4, c2):
                for u in range(4):
                    i = 4 * i4 + u
                    s = base + i
                    for d in range(_EMB // 16):
                        sl = pl.ds(d * 16, 16)
                        outb[ob, i, sl] = inb[b, i, sl] * _SCALE + pos_v[s, sl]
                return c2

            lax.fori_loop(0, _CHUNK // 4, row_body, 0)

        for b in range(_G):
            start_gather(b, b)
        for j in range(_G):
            wait_gather(j)
            if j >= 2:
                wait_scatter(j % 2)
            compute(j, j, j % 2)
            start_scatter(j, j % 2)
            start_gather(j + _G, j)

        def quad_body(q, carry):
            for b in range(_G):
                j = _G * q + b
                wait_gather(b)
                wait_scatter(b % 2)
                compute(j, b, b % 2)
                start_scatter(j, b % 2)
                start_gather(j + _G, b)
            return carry

        lax.fori_loop(1, _CHUNKS_PER_W // _G - 1, quad_body, 0)

        for b in range(_G):
            j = _CHUNKS_PER_W - _G + b
            wait_gather(b)
            wait_scatter(b % 2)
            compute(j, b, b % 2)
            start_scatter(j, b % 2)
        for b in (0, 1):
            wait_scatter(b)

    return k


_kernel_call = _make_kernel()


def kernel(x, token_table, pos_table):
    xf = x.reshape(_N // _CHUNK, _CHUNK).astype(jnp.int32)
    out = _kernel_call(xf, token_table, pos_table)
    return out.reshape(_B, _S, _EMB)
